# baseline (speedup 1.0000x reference)
import jax
import jax.numpy as jnp
from jax.experimental import pallas as pl
from jax.experimental.pallas import tpu as pltpu

T = 32768
D = 768
E = 64
TOPK = 8
N_GROUPS = 8
EPG = E // N_GROUPS
TOPK_GROUPS = 4
ROUTED_SCALING_FACTOR = 2.5

BT = 2048
NEG = -1e30


def _router_body(x_ref, w_ref, b_ref, wout_ref, iout_ref):
    x = x_ref[...]
    w = w_ref[...]
    logits = jnp.dot(x, w, preferred_element_type=jnp.float32)
    lt = jnp.transpose(logits)
    scores = jax.nn.sigmoid(lt)
    s = scores + b_ref[...]

    gs_rows = []
    for g in range(N_GROUPS):
        sg = s[EPG * g:EPG * (g + 1), :]
        t = sg + jnp.roll(sg, 1, axis=0)
        for k in (2, 3, 4):
            t = jnp.maximum(t, sg + jnp.roll(sg, k, axis=0))
        for r in (1, 2, 4):
            t = jnp.maximum(t, jnp.roll(t, r, axis=0))
        gs_rows.append(t[0:1])
    gs = jnp.concatenate(gs_rows, axis=0)

    i8g = jax.lax.broadcasted_iota(jnp.int32, gs.shape, 0).astype(jnp.float32)
    gmask = jnp.zeros(gs.shape, jnp.bool_)
    for _ in range(TOPK_GROUPS):
        m = jnp.max(gs, axis=0, keepdims=True)
        a = jnp.min(jnp.where(gs >= m, i8g, 8.0), axis=0, keepdims=True)
        hit = i8g == a
        gmask = jnp.logical_or(gmask, hit)
        gs = jnp.where(hit, NEG, gs)

    mask_e = jnp.repeat(gmask, EPG, axis=0)
    sm = jnp.where(mask_e, s, 0.0)

    i64 = jax.lax.broadcasted_iota(jnp.int32, s.shape, 0).astype(jnp.float32)
    idxs = []
    ws = []
    for _ in range(TOPK):
        m = jnp.max(sm, axis=0, keepdims=True)
        a = jnp.min(jnp.where(sm >= m, i64, 64.0), axis=0, keepdims=True)
        hit = i64 == a
        idxs.append(a)
        ws.append(jnp.sum(jnp.where(hit, scores, 0.0), axis=0, keepdims=True))
        sm = jnp.where(hit, NEG, sm)
    inds = jnp.concatenate(idxs, axis=0).astype(jnp.int32)
    w8 = jnp.concatenate(ws, axis=0)
    w8 = w8 / (jnp.sum(w8, axis=0, keepdims=True) + 1e-20)
    w8 = w8 * ROUTED_SCALING_FACTOR

    wout_ref[...] = w8
    iout_ref[...] = inds


def kernel(x_TD, kernel_DE, bias_E):
    x_TD = jnp.asarray(x_TD, jnp.float32)
    bias_col = jnp.reshape(bias_E, (E, 1))
    grid = (T // BT,)
    weights_KT, indices_KT = pl.pallas_call(
        _router_body,
        grid=grid,
        in_specs=[
            pl.BlockSpec((BT, D), lambda i: (i, 0)),
            pl.BlockSpec((D, E), lambda i: (0, 0)),
            pl.BlockSpec((E, 1), lambda i: (0, 0)),
        ],
        out_specs=[
            pl.BlockSpec((TOPK, BT), lambda i: (0, i)),
            pl.BlockSpec((TOPK, BT), lambda i: (0, i)),
        ],
        out_shape=[
            jax.ShapeDtypeStruct((TOPK, T), jnp.float32),
            jax.ShapeDtypeStruct((TOPK, T), jnp.int32),
        ],
        compiler_params=pltpu.CompilerParams(
            dimension_semantics=("arbitrary",),
        ),
    )(x_TD, kernel_DE, bias_col)
    return (jnp.transpose(weights_KT), jnp.transpose(indices_KT))
